# baseline (speedup 1.0000x reference)
import jax
import jax.numpy as jnp
from jax.experimental import pallas as pl
from jax.experimental.pallas import tpu as pltpu

PERCENT_DROP = 0.33


def _disc_kernel(drop_num, labels_ref, x_ref, w_ref, wt_ref, bias_ref,
                 y_ref, mask_ref):
    i = pl.program_id(0)
    lbl = labels_ref[i]
    xb = x_ref[0]
    C = xb.shape[0]
    hw = xb.shape[1]

    pooled = jnp.sum(xb, axis=1, keepdims=True) * (1.0 / hw)
    y = jax.lax.dot_general(pooled, w_ref[:, :], (((0,), (1,)), ((), ())),
                            preferred_element_type=jnp.float32)
    y_ref[0, 0:1, :] = y + bias_ref[0:1, :]

    nc = w_ref.shape[0]
    oh = (jax.lax.broadcasted_iota(jnp.int32, (1, nc), 1) == lbl)
    wl = jax.lax.dot_general(wt_ref[:, :], oh.astype(jnp.float32),
                             (((1,), (1,)), ((), ())),
                             preferred_element_type=jnp.float32)
    t = xb * wl
    cmax = jnp.max(t, axis=0, keepdims=True)
    cmin = jnp.min(t, axis=0, keepdims=True)
    norm = (t - cmin) / (cmax - cmin)
    cs = jnp.sum(norm, axis=1, keepdims=True) * (1.0 / hw)

    cs_row = jnp.transpose(cs)
    ii = jax.lax.broadcasted_iota(jnp.int32, (C, C), 0)
    jj = jax.lax.broadcasted_iota(jnp.int32, (C, C), 1)
    before = (cs_row > cs) | ((cs_row == cs) & (jj < ii))
    rank = jnp.sum(before.astype(jnp.float32), axis=1, keepdims=True)
    maskc = jnp.where(rank < float(drop_num), 0.0, 1.0)
    mask_ref[0, 0:1, :] = jnp.transpose(maskc)


def kernel(x, labels, W, b):
    B, C, H, Wd = x.shape
    NC = W.shape[0]
    hw = H * Wd
    drop_num = int(C * PERCENT_DROP)
    x3 = x.reshape(B, C, hw)
    labels32 = labels.astype(jnp.int32)
    WT = W.T
    b2 = b.reshape(1, NC)

    grid_spec = pltpu.PrefetchScalarGridSpec(
        num_scalar_prefetch=1,
        grid=(B,),
        in_specs=[
            pl.BlockSpec((1, C, hw), lambda i, lr: (i, 0, 0)),
            pl.BlockSpec((NC, C), lambda i, lr: (0, 0)),
            pl.BlockSpec((C, NC), lambda i, lr: (0, 0)),
            pl.BlockSpec((1, NC), lambda i, lr: (0, 0)),
        ],
        out_specs=[
            pl.BlockSpec((1, 1, NC), lambda i, lr: (i, 0, 0)),
            pl.BlockSpec((1, 1, C), lambda i, lr: (i, 0, 0)),
        ],
    )
    y, mask = pl.pallas_call(
        lambda *refs: _disc_kernel(drop_num, *refs),
        grid_spec=grid_spec,
        out_shape=[
            jax.ShapeDtypeStruct((B, 1, NC), jnp.float32),
            jax.ShapeDtypeStruct((B, 1, C), jnp.float32),
        ],
    )(labels32, x3, W, WT, b2)
    return (y.reshape(B, NC), mask.reshape(B, C, 1, 1))
